# baseline (speedup 1.0000x reference)
import functools
import jax
import jax.numpy as jnp
from jax import lax
from jax.experimental import pallas as pl
from jax.experimental.pallas import tpu as pltpu
from jax.experimental.pallas import tpu_sc as plsc

B = 2
S = 4096
D = 1024
NC, NS = 2, 16
NW = NC * NS
ROWS_W = S // NW
CH = 16
NCHUNK = ROWS_W // CH
NBUF = 2
DIST = 1

_mesh = plsc.VectorSubcoreMesh(core_axis_name="c", subcore_axis_name="s")


@functools.partial(
    pl.kernel,
    mesh=_mesh,
    out_type=jax.ShapeDtypeStruct((B, S, D), jnp.float32),
    scratch_types=[
        pltpu.VMEM((NBUF, B, CH, D), jnp.float32),
        pltpu.VMEM((NBUF, CH, D), jnp.float32),
    ] + [pltpu.SemaphoreType.DMA] * (2 * NBUF),
)
def _sc_add(x_hbm, pos_hbm, out_hbm, xbuf, pbuf, *sems):
    sin = sems[:NBUF]
    sout = sems[NBUF:]
    wid = lax.axis_index("s") * NC + lax.axis_index("c")
    row_base = wid * ROWS_W

    def issue_in(g):
        slot = g % NBUF
        r0 = row_base + g * CH
        return [
            pltpu.async_copy(pos_hbm.at[pl.ds(r0, CH), :],
                             pbuf.at[slot], sin[slot]),
            pltpu.async_copy(x_hbm.at[:, pl.ds(r0, CH), :],
                             xbuf.at[slot], sin[slot]),
        ]

    def issue_out(g):
        slot = g % NBUF
        r0 = row_base + g * CH
        return [
            pltpu.async_copy(xbuf.at[slot],
                             out_hbm.at[:, pl.ds(r0, CH), :], sout[slot]),
        ]

    def compute(g):
        slot = g % NBUF

        @plsc.parallel_loop(0, D // 16, unroll=4)
        def body(i):
            c = i * 16
            for r in range(CH):
                p = pbuf[slot, r, pl.ds(c, 16)]
                for b in range(B):
                    plsc.addupdate(xbuf.at[slot, b, r, pl.ds(c, 16)], p)

    hin = {g: issue_in(g) for g in range(min(DIST, NCHUNK))}
    hout = {}
    for g in range(NCHUNK):
        for h in hin.pop(g):
            h.wait()
        lag = g - (NBUF - DIST)
        if lag in hout:
            for h in hout.pop(lag):
                h.wait()
        if g + DIST < NCHUNK:
            hin[g + DIST] = issue_in(g + DIST)
        compute(g)
        hout[g] = issue_out(g)
    for g in list(hout):
        for h in hout.pop(g):
            h.wait()


def kernel(x, pos_table):
    return _sc_add(x, pos_table)


# speedup vs baseline: 1.0366x; 1.0366x over previous
import functools
import jax
import jax.numpy as jnp
from jax import lax
from jax.experimental import pallas as pl
from jax.experimental.pallas import tpu as pltpu
from jax.experimental.pallas import tpu_sc as plsc

B = 2
S = 4096
D = 1024
NC, NS = 2, 16
NW = NC * NS
ROWS_W = S // NW
CH = 8
NCHUNK = ROWS_W // CH
NBUF = 5
DIST = 3

_mesh = plsc.VectorSubcoreMesh(core_axis_name="c", subcore_axis_name="s")


@functools.partial(
    pl.kernel,
    mesh=_mesh,
    out_type=jax.ShapeDtypeStruct((B, S, D), jnp.float32),
    scratch_types=[
        pltpu.VMEM((NBUF, B, CH, D), jnp.float32),
        pltpu.VMEM((NBUF, CH, D), jnp.float32),
    ] + [pltpu.SemaphoreType.DMA] * (2 * NBUF),
)
def _sc_add(x_hbm, pos_hbm, out_hbm, xbuf, pbuf, *sems):
    sin = sems[:NBUF]
    sout = sems[NBUF:]
    wid = lax.axis_index("s") * NC + lax.axis_index("c")
    row_base = wid * ROWS_W

    def issue_in(g):
        slot = g % NBUF
        r0 = row_base + g * CH
        return [
            pltpu.async_copy(pos_hbm.at[pl.ds(r0, CH), :],
                             pbuf.at[slot], sin[slot]),
            pltpu.async_copy(x_hbm.at[:, pl.ds(r0, CH), :],
                             xbuf.at[slot], sin[slot]),
        ]

    def issue_out(g):
        slot = g % NBUF
        r0 = row_base + g * CH
        return [
            pltpu.async_copy(xbuf.at[slot],
                             out_hbm.at[:, pl.ds(r0, CH), :], sout[slot]),
        ]

    def compute(g):
        slot = g % NBUF

        @plsc.parallel_loop(0, D // 16, unroll=4)
        def body(i):
            c = i * 16
            for r in range(CH):
                p = pbuf[slot, r, pl.ds(c, 16)]
                for b in range(B):
                    plsc.addupdate(xbuf.at[slot, b, r, pl.ds(c, 16)], p)

    hin = {g: issue_in(g) for g in range(min(DIST, NCHUNK))}
    hout = {}
    for g in range(NCHUNK):
        for h in hin.pop(g):
            h.wait()
        lag = g - (NBUF - DIST)
        if lag in hout:
            for h in hout.pop(lag):
                h.wait()
        if g + DIST < NCHUNK:
            hin[g + DIST] = issue_in(g + DIST)
        compute(g)
        hout[g] = issue_out(g)
    for g in list(hout):
        for h in hout.pop(g):
            h.wait()


def kernel(x, pos_table):
    return _sc_add(x, pos_table)


# speedup vs baseline: 1.0374x; 1.0008x over previous
import functools
import jax
import jax.numpy as jnp
from jax import lax
from jax.experimental import pallas as pl
from jax.experimental.pallas import tpu as pltpu
from jax.experimental.pallas import tpu_sc as plsc

B = 2
S = 4096
D = 1024
NC, NS = 2, 16
NW = NC * NS
ROWS_W = S // NW
CH = 8
NCHUNK = ROWS_W // CH
NBUF = 4
DIST = 2

_mesh = plsc.VectorSubcoreMesh(core_axis_name="c", subcore_axis_name="s")


@functools.partial(
    pl.kernel,
    mesh=_mesh,
    out_type=jax.ShapeDtypeStruct((B, S, D), jnp.float32),
    scratch_types=[
        pltpu.VMEM((NBUF, B, CH, D), jnp.float32),
        pltpu.VMEM((NBUF, CH, D), jnp.float32),
    ] + [pltpu.SemaphoreType.DMA] * (2 * NBUF),
)
def _sc_add(x_hbm, pos_hbm, out_hbm, xbuf, pbuf, *sems):
    sin = sems[:NBUF]
    sout = sems[NBUF:]
    wid = lax.axis_index("s") * NC + lax.axis_index("c")
    row_base = wid * ROWS_W

    def issue_in(g):
        slot = g % NBUF
        r0 = row_base + g * CH
        return [
            pltpu.async_copy(pos_hbm.at[pl.ds(r0, CH), :],
                             pbuf.at[slot], sin[slot]),
            pltpu.async_copy(x_hbm.at[:, pl.ds(r0, CH), :],
                             xbuf.at[slot], sin[slot]),
        ]

    def issue_out(g):
        slot = g % NBUF
        r0 = row_base + g * CH
        return [
            pltpu.async_copy(xbuf.at[slot],
                             out_hbm.at[:, pl.ds(r0, CH), :], sout[slot]),
        ]

    def compute(g):
        slot = g % NBUF

        @plsc.parallel_loop(0, D // 16, unroll=4)
        def body(i):
            c = i * 16
            for r in range(CH):
                p = pbuf[slot, r, pl.ds(c, 16)]
                for b in range(B):
                    plsc.addupdate(xbuf.at[slot, b, r, pl.ds(c, 16)], p)

    hin = {g: issue_in(g) for g in range(min(DIST, NCHUNK))}
    hout = {}
    for g in range(NCHUNK):
        for h in hin.pop(g):
            h.wait()
        lag = g - (NBUF - DIST)
        if lag in hout:
            for h in hout.pop(lag):
                h.wait()
        if g + DIST < NCHUNK:
            hin[g + DIST] = issue_in(g + DIST)
        compute(g)
        hout[g] = issue_out(g)
    for g in list(hout):
        for h in hout.pop(g):
            h.wait()


def kernel(x, pos_table):
    return _sc_add(x, pos_table)


# speedup vs baseline: 1.1002x; 1.0605x over previous
import functools
import jax
import jax.numpy as jnp
from jax import lax
from jax.experimental import pallas as pl
from jax.experimental.pallas import tpu as pltpu
from jax.experimental.pallas import tpu_sc as plsc

B = 2
S = 4096
D = 1024
NC, NS = 2, 16
NW = NC * NS
ROWS_W = S // NW
CH = 8
NCHUNK = ROWS_W // CH
NBUF = 4
DIST = 2

_mesh = plsc.VectorSubcoreMesh(core_axis_name="c", subcore_axis_name="s")


@functools.partial(
    pl.kernel,
    mesh=_mesh,
    out_type=jax.ShapeDtypeStruct((B, S, D), jnp.float32),
    scratch_types=[
        pltpu.VMEM((NBUF, B, CH, D), jnp.float32),
        pltpu.VMEM((NBUF, CH, D), jnp.float32),
    ] + [pltpu.SemaphoreType.DMA] * (2 * NBUF),
)
def _sc_add(x_hbm, pos_hbm, out_hbm, xbuf, pbuf, *sems):
    sin = sems[:NBUF]
    sout = sems[NBUF:]
    wid = lax.axis_index("s") * NC + lax.axis_index("c")
    row_base = wid * ROWS_W

    def issue_in(g):
        slot = g % NBUF
        r0 = row_base + g * CH
        return [
            pltpu.async_copy(pos_hbm.at[pl.ds(r0, CH), :],
                             pbuf.at[slot], sin[slot]),
            pltpu.async_copy(x_hbm.at[:, pl.ds(r0, CH), :],
                             xbuf.at[slot], sin[slot]),
        ]

    def issue_out(g):
        slot = g % NBUF
        r0 = row_base + g * CH
        return [
            pltpu.async_copy(xbuf.at[slot],
                             out_hbm.at[:, pl.ds(r0, CH), :], sout[slot]),
        ]

    def compute(g):
        slot = g % NBUF

        @plsc.parallel_loop(0, D // 16, unroll=2)
        def body(i):
            c = i * 16
            for r in range(CH):
                p = pbuf[slot, r, pl.ds(c, 16)]
                for b in range(B):
                    plsc.addupdate(xbuf.at[slot, b, r, pl.ds(c, 16)], p)

    hin = {g: issue_in(g) for g in range(min(DIST, NCHUNK))}
    hout = {}
    for g in range(NCHUNK):
        for h in hin.pop(g):
            h.wait()
        lag = g - (NBUF - DIST)
        if lag in hout:
            for h in hout.pop(lag):
                h.wait()
        if g + DIST < NCHUNK:
            hin[g + DIST] = issue_in(g + DIST)
        compute(g)
        hout[g] = issue_out(g)
    for g in list(hout):
        for h in hout.pop(g):
            h.wait()


def kernel(x, pos_table):
    return _sc_add(x, pos_table)


# speedup vs baseline: 1.1055x; 1.0048x over previous
import functools
import jax
import jax.numpy as jnp
from jax import lax
from jax.experimental import pallas as pl
from jax.experimental.pallas import tpu as pltpu
from jax.experimental.pallas import tpu_sc as plsc

B = 2
S = 4096
D = 1024
NC, NS = 2, 16
NW = NC * NS
ROWS_W = S // NW
CH = 8
NCHUNK = ROWS_W // CH
NBUF = 4
DIST = 2

_mesh = plsc.VectorSubcoreMesh(core_axis_name="c", subcore_axis_name="s")


@functools.partial(
    pl.kernel,
    mesh=_mesh,
    out_type=jax.ShapeDtypeStruct((B, S, D), jnp.float32),
    scratch_types=[
        pltpu.VMEM((NBUF, B, CH, D), jnp.float32),
        pltpu.VMEM((NBUF, CH, D), jnp.float32),
    ] + [pltpu.SemaphoreType.DMA] * (2 * NBUF),
)
def _sc_add(x_hbm, pos_hbm, out_hbm, xbuf, pbuf, *sems):
    sin = sems[:NBUF]
    sout = sems[NBUF:]
    wid = lax.axis_index("s") * NC + lax.axis_index("c")
    row_base = wid * ROWS_W

    def issue_in(g):
        slot = g % NBUF
        r0 = row_base + g * CH
        return [
            pltpu.async_copy(pos_hbm.at[pl.ds(r0, CH), :],
                             pbuf.at[slot], sin[slot]),
            pltpu.async_copy(x_hbm.at[:, pl.ds(r0, CH), :],
                             xbuf.at[slot], sin[slot]),
        ]

    def issue_out(g):
        slot = g % NBUF
        r0 = row_base + g * CH
        return [
            pltpu.async_copy(xbuf.at[slot],
                             out_hbm.at[:, pl.ds(r0, CH), :], sout[slot]),
        ]

    def compute(g):
        slot = g % NBUF

        @plsc.parallel_loop(0, D // 16, unroll=1)
        def body(i):
            c = i * 16
            for r in range(CH):
                p = pbuf[slot, r, pl.ds(c, 16)]
                for b in range(B):
                    plsc.addupdate(xbuf.at[slot, b, r, pl.ds(c, 16)], p)

    hin = {g: issue_in(g) for g in range(min(DIST, NCHUNK))}
    hout = {}
    for g in range(NCHUNK):
        for h in hin.pop(g):
            h.wait()
        lag = g - (NBUF - DIST)
        if lag in hout:
            for h in hout.pop(lag):
                h.wait()
        if g + DIST < NCHUNK:
            hin[g + DIST] = issue_in(g + DIST)
        compute(g)
        hout[g] = issue_out(g)
    for g in list(hout):
        for h in hout.pop(g):
            h.wait()


def kernel(x, pos_table):
    return _sc_add(x, pos_table)
